# Initial kernel scaffold; baseline (speedup 1.0000x reference)
import jax
import jax.numpy as jnp
from jax.experimental import pallas as pl


def kernel(x, table):
    raise NotImplementedError("write your pallas kernel here")


# speedup vs baseline: 10.7880x; 10.7880x over previous
import functools

import jax
import jax.numpy as jnp
from jax import lax
from jax.experimental import pallas as pl
from jax.experimental.pallas import tpu as pltpu
from jax.experimental.pallas import tpu_sc as plsc

NC = 2
NS = 16
NW = NC * NS
LANES = 16

COLS_PER_TILE = 7
CHUNK = 128
INFLIGHT = 4
ROWS_PER_ROUND = CHUNK * INFLIGHT


def _sc_avg_embed(xidx, table, n_cols, batch, chunks_per_col):
  d = table.shape[1]
  n_dreg = d // LANES
  rounds = (chunks_per_col + INFLIGHT - 1) // INFLIGHT
  scale = 1.0 / batch
  mesh = plsc.VectorSubcoreMesh(core_axis_name="c", subcore_axis_name="s")

  @functools.partial(
      pl.kernel,
      mesh=mesh,
      out_type=jax.ShapeDtypeStruct((NW, COLS_PER_TILE, d), jnp.float32),
      scratch_types=[
          pltpu.VMEM((COLS_PER_TILE * chunks_per_col, CHUNK), jnp.int32),
          pltpu.VMEM((ROWS_PER_ROUND, d), jnp.float32),
          pltpu.VMEM((COLS_PER_TILE, d), jnp.float32),
          pltpu.SemaphoreType.DMA,
      ],
      compiler_params=pltpu.CompilerParams(use_tc_tiling_on_sc=False),
  )
  def body(xidx_hbm, table_hbm, out_hbm, idx_v, rows_v, stage_v, gsem):
    cid = lax.axis_index("c")
    sid = lax.axis_index("s")
    wid = sid * NC + cid
    pltpu.sync_copy(xidx_hbm.at[wid], idx_v)

    for j in range(COLS_PER_TILE):
      lcol = wid * COLS_PER_TILE + j

      @pl.when(lcol < n_cols)
      def _col():
        acc = [jnp.zeros((LANES,), jnp.float32) for _ in range(n_dreg)]

        def row_body(i, carry):
          return tuple(
              carry[r] + rows_v[i, pl.ds(r * LANES, LANES)]
              for r in range(n_dreg)
          )

        for rnd in range(rounds):
          handles = []
          for cc in range(INFLIGHT):
            ch = j * chunks_per_col + rnd * INFLIGHT + cc
            handles.append(
                pltpu.async_copy(
                    table_hbm.at[idx_v.at[ch]],
                    rows_v.at[pl.ds(cc * CHUNK, CHUNK)],
                    gsem,
                ))
          for h in handles:
            h.wait()
          acc = lax.fori_loop(0, ROWS_PER_ROUND, row_body, tuple(acc),
                              unroll=8)
          acc = list(acc)

        for r in range(n_dreg):
          stage_v[j, pl.ds(r * LANES, LANES)] = acc[r] * scale

    pltpu.sync_copy(stage_v, out_hbm.at[wid])

  return body(xidx, table)


def kernel(x, table):
  b, l = x.shape
  chunks_per_col = b // CHUNK
  l_pad = NW * COLS_PER_TILE
  xt = jnp.pad(x.T.astype(jnp.int32), ((0, l_pad - l), (0, 0)))
  xidx = xt.reshape(NW, COLS_PER_TILE * chunks_per_col, CHUNK)
  out = _sc_avg_embed(xidx, table, l, b, chunks_per_col)
  return out.reshape(l_pad, -1)[:l]


# speedup vs baseline: 16.1233x; 1.4946x over previous
import functools

import jax
import jax.numpy as jnp
from jax import lax
from jax.experimental import pallas as pl
from jax.experimental.pallas import tpu as pltpu
from jax.experimental.pallas import tpu_sc as plsc

NC = 2
NS = 16
NW = NC * NS
LANES = 16

COLS_PER_TILE = 7
CHUNK = 128
INFLIGHT = 4
RPB = CHUNK * INFLIGHT


def _sc_avg_embed(xidx, table, n_cols, batch, chunks_per_col):
  d = table.shape[1]
  n_dreg = d // LANES
  rounds_per_col = chunks_per_col // INFLIGHT
  scale = 1.0 / batch
  mesh = plsc.VectorSubcoreMesh(core_axis_name="c", subcore_axis_name="s")

  @functools.partial(
      pl.kernel,
      mesh=mesh,
      out_type=jax.ShapeDtypeStruct((NW, COLS_PER_TILE, d), jnp.float32),
      scratch_types=[
          pltpu.VMEM((COLS_PER_TILE * chunks_per_col, CHUNK), jnp.int32),
          pltpu.VMEM((2 * RPB, d), jnp.float32),
          pltpu.VMEM((COLS_PER_TILE, d), jnp.float32),
          pltpu.SemaphoreType.DMA,
          pltpu.SemaphoreType.DMA,
      ],
      compiler_params=pltpu.CompilerParams(use_tc_tiling_on_sc=False),
  )
  def body(xidx_hbm, table_hbm, out_hbm, idx_v, rows_v, stage_v, sem0, sem1):
    cid = lax.axis_index("c")
    sid = lax.axis_index("s")
    wid = sid * NC + cid
    pltpu.sync_copy(xidx_hbm.at[wid], idx_v)

    nv = lax.max(0, lax.min(COLS_PER_TILE, n_cols - wid * COLS_PER_TILE))
    nrounds = nv * rounds_per_col
    sems = (sem0, sem1)

    def fire(rnd, parity):
      base = parity * RPB
      for cc in range(INFLIGHT):
        pltpu.async_copy(
            table_hbm.at[idx_v.at[rnd * INFLIGHT + cc]],
            rows_v.at[pl.ds(base + cc * CHUNK, CHUNK)],
            sems[parity],
        )

    def do_round(rnd, parity, acc):
      base = parity * RPB

      @pl.when(rnd + 1 < nrounds)
      def _fire_next():
        fire(rnd + 1, 1 - parity)

      pltpu.make_async_copy(
          table_hbm.at[pl.ds(0, RPB)],
          rows_v.at[pl.ds(base, RPB)],
          sems[parity],
      ).wait()

      def row_body(i, carry):
        return tuple(
            carry[r] + rows_v[base + i, pl.ds(r * LANES, LANES)]
            for r in range(n_dreg)
        )

      acc = lax.fori_loop(0, RPB, row_body, acc, unroll=8)

      col = rnd // rounds_per_col
      done = lax.rem(rnd, rounds_per_col) == (rounds_per_col - 1)

      @pl.when(done)
      def _store():
        for r in range(n_dreg):
          stage_v[col, pl.ds(r * LANES, LANES)] = acc[r] * scale

      return tuple(jnp.where(done, jnp.zeros_like(a), a) for a in acc)

    @pl.when(nrounds > 0)
    def _prologue():
      fire(0, 0)

    def pair_body(rp, acc):
      acc = do_round(2 * rp, 0, acc)
      acc = do_round(2 * rp + 1, 1, acc)
      return acc

    zeros = tuple(jnp.zeros((LANES,), jnp.float32) for _ in range(n_dreg))
    lax.fori_loop(0, nrounds // 2, pair_body, zeros)

    pltpu.sync_copy(stage_v, out_hbm.at[wid])

  return body(xidx, table)


def kernel(x, table):
  b, l = x.shape
  chunks_per_col = b // CHUNK
  l_pad = NW * COLS_PER_TILE
  xt = jnp.pad(x.T.astype(jnp.int32), ((0, l_pad - l), (0, 0)))
  xidx = xt.reshape(NW, COLS_PER_TILE * chunks_per_col, CHUNK)
  out = _sc_avg_embed(xidx, table, l, b, chunks_per_col)
  return out.reshape(l_pad, -1)[:l]
